# baseline (speedup 1.0000x reference)
import jax
import jax.numpy as jnp
from jax.experimental import pallas as pl
from jax.experimental.pallas import tpu as pltpu

EPS = 1e-10
_FBLK = 57
_TC = 2048


def _body(pred_ref, tar_ref, mask_ref, out_ref, s_acc, m_f32):
    i = pl.program_id(0)
    B = mask_ref.shape[0]
    T = mask_ref.shape[1]
    F = pl.num_programs(0) * _FBLK

    @pl.when(i == 0)
    def _():
        s_acc[...] = jnp.zeros_like(s_acc)
        m_f32[...] = mask_ref[...].astype(jnp.float32)

    for tc in range(T // _TC):
        sl = pl.ds(tc * _TC, _TC)
        m = m_f32[:, sl]

        def f_body(f, acc):
            p = pred_ref[f, :, sl]
            y = tar_ref[f, :, sl]
            return acc + jnp.abs(p - jnp.log(y + EPS)) * m

        acc = jax.lax.fori_loop(0, _FBLK, f_body, jnp.zeros((B, _TC), jnp.float32))
        s_acc[:, sl] += acc

    @pl.when(i == pl.num_programs(0) - 1)
    def _():
        out_ref[...] = (jnp.sum(s_acc[...]) / (jnp.sum(m_f32[...]) * F)).reshape(1, 1)


def kernel(log_predicted, linear_tar, stft_length_masks):
    B, T, F = log_predicted.shape
    pred_t = jnp.transpose(log_predicted, (2, 0, 1))
    tar_t = jnp.transpose(linear_tar, (2, 0, 1))

    out = pl.pallas_call(
        _body,
        grid=(F // _FBLK,),
        in_specs=[
            pl.BlockSpec((_FBLK, B, T), lambda i: (i, 0, 0)),
            pl.BlockSpec((_FBLK, B, T), lambda i: (i, 0, 0)),
            pl.BlockSpec((B, T), lambda i: (0, 0)),
        ],
        out_specs=pl.BlockSpec((1, 1), lambda i: (0, 0)),
        out_shape=jax.ShapeDtypeStruct((1, 1), jnp.float32),
        scratch_shapes=[
            pltpu.VMEM((B, T), jnp.float32),
            pltpu.VMEM((B, T), jnp.float32),
        ],
    )(pred_t, tar_t, stft_length_masks)
    return out[0, 0]
